# Initial kernel scaffold; baseline (speedup 1.0000x reference)
import functools

import jax
import jax.numpy as jnp
from jax import lax
from jax.experimental import pallas as pl
from jax.experimental.pallas import tpu as pltpu
from jax.experimental.pallas import tpu_sc as plsc

B = 16384
F = 26
V = 100000
D = 40
ND = 13
H = 64
TOT = B * F
NW = 32
PER_W = TOT // NW
CH = 784
NCH = PER_W // CH

_EPS = 1e-7

_sc_mesh = plsc.VectorSubcoreMesh(core_axis_name="c", subcore_axis_name="s")


@functools.partial(
    pl.kernel,
    mesh=_sc_mesh,
    out_type=[
        jax.ShapeDtypeStruct((TOT, D), jnp.float32),
        jax.ShapeDtypeStruct((TOT, 1), jnp.float32),
    ],
    scratch_types=[
        pltpu.VMEM((CH,), jnp.int32),
        pltpu.VMEM((CH, D), jnp.float32),
        pltpu.VMEM((CH, 1), jnp.float32),
        pltpu.SemaphoreType.DMA,
        pltpu.SemaphoreType.DMA,
    ],
)
def _sc_gather(emb_hbm, wide_hbm, idx_hbm, out_d, out_w,
               idx_v, rows_v, wrows_v, sem1, sem2):
    wid = lax.axis_index("s") * 2 + lax.axis_index("c")
    base = wid * PER_W

    def body(j, carry):
        off = base + j * CH
        pltpu.sync_copy(idx_hbm.at[pl.ds(off, CH)], idx_v)
        cp1 = pltpu.async_copy(emb_hbm.at[idx_v], rows_v, sem1)
        cp2 = pltpu.async_copy(wide_hbm.at[idx_v], wrows_v, sem2)
        cp1.wait()
        cp2.wait()
        pltpu.sync_copy(rows_v, out_d.at[pl.ds(off, CH)])
        pltpu.sync_copy(wrows_v, out_w.at[pl.ds(off, CH)])
        return carry

    lax.fori_loop(0, NCH, body, 0)


def _mlp_body(semb, wemb, dense, ylab,
              W1s, W1d, b1, W2, b2, W3, b3, Wo, bo, Wws, Wwd, bw,
              ypred, loss, acc):
    i = pl.program_id(0)
    x = semb[...]
    dd = dense[...]
    h = jnp.maximum(
        jnp.dot(x, W1s[...], preferred_element_type=jnp.float32)
        + jnp.dot(dd, W1d[...], preferred_element_type=jnp.float32)
        + b1[...], 0.0)
    h = jnp.maximum(
        jnp.dot(h, W2[...], preferred_element_type=jnp.float32) + b2[...], 0.0)
    h = jnp.maximum(
        jnp.dot(h, W3[...], preferred_element_type=jnp.float32) + b3[...], 0.0)
    deep = jax.nn.sigmoid(
        jnp.sum(h * Wo[...], axis=1, keepdims=True) + bo[...])
    wide = (jnp.sum(wemb[...] * Wws[...], axis=1, keepdims=True)
            + jnp.sum(dd * Wwd[...], axis=1, keepdims=True) + bw[...])
    y = jax.nn.sigmoid(wide + deep)
    ypred[...] = y
    p = jnp.clip(y, _EPS, 1.0 - _EPS)
    yl = ylab[...]
    s = jnp.sum(yl * jnp.log(p) + (1.0 - yl) * jnp.log(1.0 - p))
    total = jnp.where(i == 0, 0.0, acc[0]) + s
    acc[0] = total

    @pl.when(i == pl.num_programs(0) - 1)
    def _():
        loss[...] = jnp.full((1, 1), -total / B, jnp.float32)


BLK = 1024


def _mlp_call(semb, wemb, dense, ylab, W1s, W1d, b1, W2, b2, W3, b3,
              Wo, bo, Wws, Wwd, bw, interpret=False):
    grid = (B // BLK,)
    row = lambda i: (i, 0)
    fixed = lambda i: (0, 0)
    return pl.pallas_call(
        _mlp_body,
        grid=grid,
        in_specs=[
            pl.BlockSpec((BLK, F * D), row),
            pl.BlockSpec((BLK, F), row),
            pl.BlockSpec((BLK, ND), row),
            pl.BlockSpec((BLK, 1), row),
            pl.BlockSpec((F * D, H), fixed),
            pl.BlockSpec((ND, H), fixed),
            pl.BlockSpec((1, H), fixed),
            pl.BlockSpec((H, H), fixed),
            pl.BlockSpec((1, H), fixed),
            pl.BlockSpec((H, H), fixed),
            pl.BlockSpec((1, H), fixed),
            pl.BlockSpec((1, H), fixed),
            pl.BlockSpec((1, 1), fixed),
            pl.BlockSpec((1, F), fixed),
            pl.BlockSpec((1, ND), fixed),
            pl.BlockSpec((1, 1), fixed),
        ],
        out_specs=[
            pl.BlockSpec((BLK, 1), row),
            pl.BlockSpec((1, 1), fixed),
        ],
        out_shape=[
            jax.ShapeDtypeStruct((B, 1), jnp.float32),
            jax.ShapeDtypeStruct((1, 1), jnp.float32),
        ],
        scratch_shapes=[pltpu.SMEM((1,), jnp.float32)],
        interpret=interpret,
    )(semb, wemb, dense, ylab, W1s, W1d, b1, W2, b2, W3, b3,
      Wo, bo, Wws, Wwd, bw)


def kernel(sparse_ids, dense_feats, label, emb_table, wide_table,
           Ww, bw, W1, b1, W2, b2, W3, b3, Wo, bo):
    offsets = (jnp.arange(F, dtype=sparse_ids.dtype) * V)[None, :]
    flat_ids = (sparse_ids + offsets).reshape(TOT)
    semb_flat, wemb_flat = _sc_gather(emb_table, wide_table, flat_ids)
    semb = semb_flat.reshape(B, F * D)
    wemb = wemb_flat.reshape(B, F)
    ylab = label.astype(jnp.float32).reshape(B, 1)
    y_pred, loss = _mlp_call(
        semb, wemb, dense_feats, ylab,
        W1[:F * D], W1[F * D:], b1.reshape(1, H),
        W2, b2.reshape(1, H), W3, b3.reshape(1, H),
        Wo.reshape(1, H), bo.reshape(1, 1),
        Ww[:F].reshape(1, F), Ww[F:].reshape(1, ND), bw.reshape(1, 1))
    return y_pred, loss.reshape(())


# speedup vs baseline: 2.0317x; 2.0317x over previous
import functools

import jax
import jax.numpy as jnp
from jax import lax
from jax.experimental import pallas as pl
from jax.experimental.pallas import tpu as pltpu
from jax.experimental.pallas import tpu_sc as plsc

B = 16384
F = 26
V = 100000
D = 40
ND = 13
H = 64
TOT = B * F
NW = 32
PER_W = TOT // NW
CH = 784
NCH = PER_W // CH

_EPS = 1e-7

@functools.cache
def _build_sc_gather():
    mesh = plsc.VectorSubcoreMesh(core_axis_name="c", subcore_axis_name="s")

    @functools.partial(
        pl.kernel,
        mesh=mesh,
        out_type=[
            jax.ShapeDtypeStruct((TOT, D), jnp.float32),
            jax.ShapeDtypeStruct((TOT, 1), jnp.float32),
        ],
        scratch_types=[
            pltpu.VMEM((CH,), jnp.int32),
            pltpu.VMEM((CH, D), jnp.float32),
            pltpu.VMEM((CH, 1), jnp.float32),
            pltpu.SemaphoreType.DMA,
            pltpu.SemaphoreType.DMA,
        ],
        compiler_params=pltpu.CompilerParams(use_tc_tiling_on_sc=False),
    )
    def _sc_gather(emb_hbm, wide_hbm, idx_hbm, out_d, out_w,
                   idx_v, rows_v, wrows_v, sem1, sem2):
        wid = lax.axis_index("s") * 2 + lax.axis_index("c")
        base = wid * PER_W

        def body(j, carry):
            off = base + j * CH
            pltpu.sync_copy(idx_hbm.at[pl.ds(off, CH)], idx_v)
            cp1 = pltpu.async_copy(emb_hbm.at[idx_v], rows_v, sem1)
            cp2 = pltpu.async_copy(wide_hbm.at[idx_v], wrows_v, sem2)
            cp1.wait()
            cp2.wait()
            pltpu.sync_copy(rows_v, out_d.at[pl.ds(off, CH)])
            pltpu.sync_copy(wrows_v, out_w.at[pl.ds(off, CH)])
            return carry

        lax.fori_loop(0, NCH, body, 0)

    return _sc_gather


def _mlp_body(semb, wemb, dense, ylab,
              W1s, W1d, b1, W2, b2, W3, b3, Wo, bo, Wws, Wwd, bw,
              ypred, loss, acc):
    i = pl.program_id(0)
    x = semb[...]
    dd = dense[...]
    h = jnp.maximum(
        jnp.dot(x, W1s[...], preferred_element_type=jnp.float32)
        + jnp.dot(dd, W1d[...], preferred_element_type=jnp.float32)
        + b1[...], 0.0)
    h = jnp.maximum(
        jnp.dot(h, W2[...], preferred_element_type=jnp.float32) + b2[...], 0.0)
    h = jnp.maximum(
        jnp.dot(h, W3[...], preferred_element_type=jnp.float32) + b3[...], 0.0)
    deep = jax.nn.sigmoid(
        jnp.sum(h * Wo[...], axis=1, keepdims=True) + bo[...])
    wide = (jnp.sum(wemb[...] * Wws[...], axis=1, keepdims=True)
            + jnp.sum(dd * Wwd[...], axis=1, keepdims=True) + bw[...])
    y = jax.nn.sigmoid(wide + deep)
    ypred[...] = y
    p = jnp.clip(y, _EPS, 1.0 - _EPS)
    yl = ylab[...]
    s = jnp.sum(yl * jnp.log(p) + (1.0 - yl) * jnp.log(1.0 - p))
    total = jnp.where(i == 0, 0.0, acc[0]) + s
    acc[0] = total

    @pl.when(i == pl.num_programs(0) - 1)
    def _():
        loss[...] = jnp.full((1, 1), -total / B, jnp.float32)


BLK = 1024


def _mlp_call(semb, wemb, dense, ylab, W1s, W1d, b1, W2, b2, W3, b3,
              Wo, bo, Wws, Wwd, bw):
    grid = (B // BLK,)
    row = lambda i: (i, 0)
    fixed = lambda i: (0, 0)
    return pl.pallas_call(
        _mlp_body,
        grid=grid,
        in_specs=[
            pl.BlockSpec((BLK, F * D), row),
            pl.BlockSpec((BLK, F), row),
            pl.BlockSpec((BLK, ND), row),
            pl.BlockSpec((BLK, 1), row),
            pl.BlockSpec((F * D, H), fixed),
            pl.BlockSpec((ND, H), fixed),
            pl.BlockSpec((1, H), fixed),
            pl.BlockSpec((H, H), fixed),
            pl.BlockSpec((1, H), fixed),
            pl.BlockSpec((H, H), fixed),
            pl.BlockSpec((1, H), fixed),
            pl.BlockSpec((1, H), fixed),
            pl.BlockSpec((1, 1), fixed),
            pl.BlockSpec((1, F), fixed),
            pl.BlockSpec((1, ND), fixed),
            pl.BlockSpec((1, 1), fixed),
        ],
        out_specs=[
            pl.BlockSpec((BLK, 1), row),
            pl.BlockSpec((1, 1), fixed),
        ],
        out_shape=[
            jax.ShapeDtypeStruct((B, 1), jnp.float32),
            jax.ShapeDtypeStruct((1, 1), jnp.float32),
        ],
        scratch_shapes=[pltpu.SMEM((1,), jnp.float32)],
    )(semb, wemb, dense, ylab, W1s, W1d, b1, W2, b2, W3, b3,
      Wo, bo, Wws, Wwd, bw)


def kernel(sparse_ids, dense_feats, label, emb_table, wide_table,
           Ww, bw, W1, b1, W2, b2, W3, b3, Wo, bo):
    offsets = (jnp.arange(F, dtype=sparse_ids.dtype) * V)[None, :]
    flat_ids = (sparse_ids + offsets).reshape(TOT)
    semb_flat, wemb_flat = _build_sc_gather()(emb_table, wide_table, flat_ids)
    semb = semb_flat.reshape(B, F * D)
    wemb = wemb_flat.reshape(B, F)
    ylab = label.astype(jnp.float32).reshape(B, 1)
    y_pred, loss = _mlp_call(
        semb, wemb, dense_feats, ylab,
        W1[:F * D], W1[F * D:], b1.reshape(1, H),
        W2, b2.reshape(1, H), W3, b3.reshape(1, H),
        Wo.reshape(1, H), bo.reshape(1, 1),
        Ww[:F].reshape(1, F), Ww[F:].reshape(1, ND), bw.reshape(1, 1))
    return y_pred, loss.reshape(())


# speedup vs baseline: 3.9738x; 1.9559x over previous
import functools

import jax
import jax.numpy as jnp
from jax import lax
from jax.experimental import pallas as pl
from jax.experimental.pallas import tpu as pltpu
from jax.experimental.pallas import tpu_sc as plsc

B = 16384
F = 26
V = 100000
D = 40
ND = 13
H = 64
TOT = B * F
NW = 32
PER_W = TOT // NW
CH = 832
NCH = PER_W // CH
assert CH * NCH == PER_W and PER_W * NW == TOT and CH % 8 == 0

_EPS = 1e-7

@functools.cache
def _build_sc_gather():
    mesh = plsc.VectorSubcoreMesh(core_axis_name="c", subcore_axis_name="s")

    @functools.partial(
        pl.kernel,
        mesh=mesh,
        out_type=[
            jax.ShapeDtypeStruct((TOT, D), jnp.float32),
            jax.ShapeDtypeStruct((TOT,), jnp.float32),
        ],
        scratch_types=[
            pltpu.VMEM((CH,), jnp.int32),
            pltpu.VMEM((CH, D), jnp.float32),
            pltpu.VMEM((CH,), jnp.float32),
            pltpu.SemaphoreType.DMA,
            pltpu.SemaphoreType.DMA,
        ],
        compiler_params=pltpu.CompilerParams(use_tc_tiling_on_sc=False),
    )
    def _sc_gather(emb_hbm, wide_hbm, idx_hbm, out_d, out_w,
                   idx_v, rows_v, wrows_v, sem1, sem2):
        wid = lax.axis_index("s") * 2 + lax.axis_index("c")
        base = wid * PER_W

        def body(j, carry):
            off = base + j * CH
            pltpu.sync_copy(idx_hbm.at[pl.ds(off, CH)], idx_v)
            cp1 = pltpu.async_copy(emb_hbm.at[idx_v], rows_v, sem1)
            cp2 = pltpu.async_copy(wide_hbm.at[idx_v], wrows_v, sem2)
            cp1.wait()
            cp2.wait()
            pltpu.sync_copy(rows_v, out_d.at[pl.ds(off, CH)])
            pltpu.sync_copy(wrows_v, out_w.at[pl.ds(off, CH)])
            return carry

        lax.fori_loop(0, NCH, body, 0)

    return _sc_gather


def _mlp_body(semb, wemb, dense, ylab,
              W1s, W1d, b1, W2, b2, W3, b3, Wo, bo, Wws, Wwd, bw,
              ypred, loss, acc):
    i = pl.program_id(0)
    x = semb[...]
    dd = dense[...]
    h = jnp.maximum(
        jnp.dot(x, W1s[...], preferred_element_type=jnp.float32)
        + jnp.dot(dd, W1d[...], preferred_element_type=jnp.float32)
        + b1[...], 0.0)
    h = jnp.maximum(
        jnp.dot(h, W2[...], preferred_element_type=jnp.float32) + b2[...], 0.0)
    h = jnp.maximum(
        jnp.dot(h, W3[...], preferred_element_type=jnp.float32) + b3[...], 0.0)
    deep = jax.nn.sigmoid(
        jnp.sum(h * Wo[...], axis=1, keepdims=True) + bo[...])
    wide = (jnp.sum(wemb[...] * Wws[...], axis=1, keepdims=True)
            + jnp.sum(dd * Wwd[...], axis=1, keepdims=True) + bw[...])
    y = jax.nn.sigmoid(wide + deep)
    ypred[...] = y
    p = jnp.clip(y, _EPS, 1.0 - _EPS)
    yl = ylab[...]
    s = jnp.sum(yl * jnp.log(p) + (1.0 - yl) * jnp.log(1.0 - p))
    total = jnp.where(i == 0, 0.0, acc[0]) + s
    acc[0] = total

    @pl.when(i == pl.num_programs(0) - 1)
    def _():
        loss[...] = jnp.full((1, 1), -total / B, jnp.float32)


BLK = 1024


def _mlp_call(semb, wemb, dense, ylab, W1s, W1d, b1, W2, b2, W3, b3,
              Wo, bo, Wws, Wwd, bw):
    grid = (B // BLK,)
    row = lambda i: (i, 0)
    fixed = lambda i: (0, 0)
    return pl.pallas_call(
        _mlp_body,
        grid=grid,
        in_specs=[
            pl.BlockSpec((BLK, F * D), row),
            pl.BlockSpec((BLK, F), row),
            pl.BlockSpec((BLK, ND), row),
            pl.BlockSpec((BLK, 1), row),
            pl.BlockSpec((F * D, H), fixed),
            pl.BlockSpec((ND, H), fixed),
            pl.BlockSpec((1, H), fixed),
            pl.BlockSpec((H, H), fixed),
            pl.BlockSpec((1, H), fixed),
            pl.BlockSpec((H, H), fixed),
            pl.BlockSpec((1, H), fixed),
            pl.BlockSpec((1, H), fixed),
            pl.BlockSpec((1, 1), fixed),
            pl.BlockSpec((1, F), fixed),
            pl.BlockSpec((1, ND), fixed),
            pl.BlockSpec((1, 1), fixed),
        ],
        out_specs=[
            pl.BlockSpec((BLK, 1), row),
            pl.BlockSpec((1, 1), fixed),
        ],
        out_shape=[
            jax.ShapeDtypeStruct((B, 1), jnp.float32),
            jax.ShapeDtypeStruct((1, 1), jnp.float32),
        ],
        scratch_shapes=[pltpu.SMEM((1,), jnp.float32)],
    )(semb, wemb, dense, ylab, W1s, W1d, b1, W2, b2, W3, b3,
      Wo, bo, Wws, Wwd, bw)


def kernel(sparse_ids, dense_feats, label, emb_table, wide_table,
           Ww, bw, W1, b1, W2, b2, W3, b3, Wo, bo):
    offsets = (jnp.arange(F, dtype=sparse_ids.dtype) * V)[None, :]
    flat_ids = (sparse_ids + offsets).reshape(TOT)
    emb_lin = lax.optimization_barrier(emb_table.reshape(-1)).reshape(F * V, D)
    wide_lin = wide_table.reshape(-1)
    semb_flat, wemb_flat = _build_sc_gather()(emb_lin, wide_lin, flat_ids)
    semb = semb_flat.reshape(B, F * D)
    wemb = wemb_flat.reshape(B, F)
    ylab = label.astype(jnp.float32).reshape(B, 1)
    y_pred, loss = _mlp_call(
        semb, wemb, dense_feats, ylab,
        W1[:F * D], W1[F * D:], b1.reshape(1, H),
        W2, b2.reshape(1, H), W3, b3.reshape(1, H),
        Wo.reshape(1, H), bo.reshape(1, 1),
        Ww[:F].reshape(1, F), Ww[F:].reshape(1, ND), bw.reshape(1, 1))
    return y_pred, loss.reshape(())


# speedup vs baseline: 9.7135x; 2.4444x over previous
import functools

import jax
import jax.numpy as jnp
from jax import lax
from jax.experimental import pallas as pl
from jax.experimental.pallas import tpu as pltpu
from jax.experimental.pallas import tpu_sc as plsc

B = 16384
F = 26
V = 100000
D = 40
ND = 13
H = 64
TOT = B * F
NW = 32
PER_W = TOT // NW
CH = 832
NCH = PER_W // CH
assert CH * NCH == PER_W and PER_W * NW == TOT and CH % 8 == 0

_EPS = 1e-7

@functools.cache
def _build_sc_gather():
    mesh = plsc.VectorSubcoreMesh(core_axis_name="c", subcore_axis_name="s")

    @functools.partial(
        pl.kernel,
        mesh=mesh,
        out_type=[
            jax.ShapeDtypeStruct((TOT, D), jnp.float32),
            jax.ShapeDtypeStruct((TOT,), jnp.float32),
        ],
        scratch_types=[
            pltpu.VMEM((CH,), jnp.int32),
            pltpu.VMEM((CH, DPAD), jnp.float32),
            pltpu.VMEM((CH,), jnp.float32),
            pltpu.SemaphoreType.DMA,
            pltpu.SemaphoreType.DMA,
        ],
        compiler_params=pltpu.CompilerParams(use_tc_tiling_on_sc=False),
    )
    def _sc_gather(emb_hbm, wide_hbm, idx_hbm, out_d, out_w,
                   idx_v, rows_v, wrows_v, sem1, sem2):
        wid = lax.axis_index("s") * 2 + lax.axis_index("c")
        base = wid * PER_W

        def body(j, carry):
            off = base + j * CH
            pltpu.sync_copy(idx_hbm.at[pl.ds(off, CH)], idx_v)
            cp1 = pltpu.async_copy(emb_hbm.at[idx_v], rows_v, sem1)
            cp2 = pltpu.async_copy(wide_hbm.at[idx_v], wrows_v, sem2)
            cp1.wait()
            cp2.wait()
            pltpu.sync_copy(rows_v.at[:, pl.ds(0, D)],
                            out_d.at[pl.ds(off, CH)])
            pltpu.sync_copy(wrows_v, out_w.at[pl.ds(off, CH)])
            return carry

        lax.fori_loop(0, NCH, body, 0)

    return _sc_gather


TBLK = 4096
DPAD = 128


def _detile_body(src, dst):
    z = jnp.transpose(src[...])
    dst[...] = jnp.concatenate(
        [z, jnp.zeros((TBLK, DPAD - D), jnp.float32)], axis=1)


def _detile(emb_t):
    n = F * V
    return pl.pallas_call(
        _detile_body,
        grid=(pl.cdiv(n, TBLK),),
        in_specs=[pl.BlockSpec((D, TBLK), lambda i: (0, i))],
        out_specs=pl.BlockSpec((TBLK, DPAD), lambda i: (i, 0)),
        out_shape=jax.ShapeDtypeStruct((n, DPAD), jnp.float32),
    )(emb_t)


def _mlp_body(semb, wemb, dense, ylab,
              W1s, W1d, b1, W2, b2, W3, b3, Wo, bo, Wws, Wwd, bw,
              ypred, loss, acc):
    i = pl.program_id(0)
    x = semb[...]
    dd = dense[...]
    h = jnp.maximum(
        jnp.dot(x, W1s[...], preferred_element_type=jnp.float32)
        + jnp.dot(dd, W1d[...], preferred_element_type=jnp.float32)
        + b1[...], 0.0)
    h = jnp.maximum(
        jnp.dot(h, W2[...], preferred_element_type=jnp.float32) + b2[...], 0.0)
    h = jnp.maximum(
        jnp.dot(h, W3[...], preferred_element_type=jnp.float32) + b3[...], 0.0)
    deep = jax.nn.sigmoid(
        jnp.sum(h * Wo[...], axis=1, keepdims=True) + bo[...])
    wide = (jnp.sum(wemb[...] * Wws[...], axis=1, keepdims=True)
            + jnp.sum(dd * Wwd[...], axis=1, keepdims=True) + bw[...])
    y = jax.nn.sigmoid(wide + deep)
    ypred[...] = y
    p = jnp.clip(y, _EPS, 1.0 - _EPS)
    yl = ylab[...]
    s = jnp.sum(yl * jnp.log(p) + (1.0 - yl) * jnp.log(1.0 - p))
    total = jnp.where(i == 0, 0.0, acc[0]) + s
    acc[0] = total

    @pl.when(i == pl.num_programs(0) - 1)
    def _():
        loss[...] = jnp.full((1, 1), -total / B, jnp.float32)


BLK = 1024


def _mlp_call(semb, wemb, dense, ylab, W1s, W1d, b1, W2, b2, W3, b3,
              Wo, bo, Wws, Wwd, bw):
    grid = (B // BLK,)
    row = lambda i: (i, 0)
    fixed = lambda i: (0, 0)
    return pl.pallas_call(
        _mlp_body,
        grid=grid,
        in_specs=[
            pl.BlockSpec((BLK, F * D), row),
            pl.BlockSpec((BLK, F), row),
            pl.BlockSpec((BLK, ND), row),
            pl.BlockSpec((BLK, 1), row),
            pl.BlockSpec((F * D, H), fixed),
            pl.BlockSpec((ND, H), fixed),
            pl.BlockSpec((1, H), fixed),
            pl.BlockSpec((H, H), fixed),
            pl.BlockSpec((1, H), fixed),
            pl.BlockSpec((H, H), fixed),
            pl.BlockSpec((1, H), fixed),
            pl.BlockSpec((1, H), fixed),
            pl.BlockSpec((1, 1), fixed),
            pl.BlockSpec((1, F), fixed),
            pl.BlockSpec((1, ND), fixed),
            pl.BlockSpec((1, 1), fixed),
        ],
        out_specs=[
            pl.BlockSpec((BLK, 1), row),
            pl.BlockSpec((1, 1), fixed),
        ],
        out_shape=[
            jax.ShapeDtypeStruct((B, 1), jnp.float32),
            jax.ShapeDtypeStruct((1, 1), jnp.float32),
        ],
        scratch_shapes=[pltpu.SMEM((1,), jnp.float32)],
    )(semb, wemb, dense, ylab, W1s, W1d, b1, W2, b2, W3, b3,
      Wo, bo, Wws, Wwd, bw)


def kernel(sparse_ids, dense_feats, label, emb_table, wide_table,
           Ww, bw, W1, b1, W2, b2, W3, b3, Wo, bo):
    offsets = (jnp.arange(F, dtype=sparse_ids.dtype) * V)[None, :]
    flat_ids = (sparse_ids + offsets).reshape(TOT)
    emb_lin = _detile(emb_table.T)
    wide_lin = wide_table.reshape(-1)
    semb_flat, wemb_flat = _build_sc_gather()(emb_lin, wide_lin, flat_ids)
    semb = semb_flat.reshape(B, F * D)
    wemb = wemb_flat.reshape(B, F)
    ylab = label.astype(jnp.float32).reshape(B, 1)
    y_pred, loss = _mlp_call(
        semb, wemb, dense_feats, ylab,
        W1[:F * D], W1[F * D:], b1.reshape(1, H),
        W2, b2.reshape(1, H), W3, b3.reshape(1, H),
        Wo.reshape(1, H), bo.reshape(1, 1),
        Ww[:F].reshape(1, F), Ww[F:].reshape(1, ND), bw.reshape(1, 1))
    return y_pred, loss.reshape(())
